# baseline (speedup 1.0000x reference)
import jax
import jax.numpy as jnp
from jax.experimental import pallas as pl

BATCH = 16
NUM_POINTS = 128
C = 16
OUT = 64
NCLASS = 15
R = 16
DATT = 16
NG = BATCH * NUM_POINTS
NS = 16384
NR = 1024


def _rbf_expand(r):
    centers = jnp.linspace(0.0, 4.0, R)
    return jnp.exp(-4.0 * (r[:, None] - centers[None, :]) ** 2)


def _segment_softmax(logits, seg, n):
    m = jax.ops.segment_max(logits, seg, num_segments=n)
    m = jnp.where(jnp.isfinite(m), m, 0.0)
    e = jnp.exp(logits - m[seg])
    s = jax.ops.segment_sum(e, seg, num_segments=n)
    return e / (s[seg] + 1e-9)


def _se3_layer(h0, h1, src, dst, rbf, rel, p, n):
    k = rbf @ p['Wk_r'] + h0[src] @ p['Wk_h']
    q = (h0 @ p['Wq'])[dst]
    logits = jnp.sum(q * k, axis=-1) / jnp.sqrt(float(DATT))
    alpha = _segment_softmax(logits, dst, n)
    v0 = rbf @ p['Wv0_r'] + h0[src] @ p['Wv0_h']
    v1 = jnp.einsum('eci,co->eoi', h1[src], p['Wv1_mix']) + (rbf @ p['Wv1_r'])[:, :, None] * rel[:, None, :]
    m0 = jax.ops.segment_sum(alpha[:, None] * v0, dst, num_segments=n)
    m1 = jax.ops.segment_sum(alpha[:, None, None] * v1, dst, num_segments=n)
    h0o = m0 + h0 @ p['Ws0']
    h1o = m1 + jnp.einsum('nci,co->noi', h1, p['Ws1'])
    h0o = jax.nn.relu(h0o + p['b0'])
    nrm = jnp.sqrt(jnp.sum(h1o ** 2, axis=-1, keepdims=True) + 1e-8)
    h1o = h1o * (jax.nn.relu(nrm + p['b1'][None, :, None]) / nrm)
    return h0o, h1o


def _run_block(x, edge_index, h1_in, plist, n):
    src, dst = edge_index[0], edge_index[1]
    rel = x[dst] - x[src]
    r = jnp.sqrt(jnp.sum(rel ** 2, axis=-1) + 1e-8)
    rbf = _rbf_expand(r)
    h0 = jnp.zeros((n, 1), x.dtype)
    h1 = h1_in
    for p in plist:
        h0, h1 = _se3_layer(h0, h1, src, dst, rbf, rel, p, n)
    return h0, h1


def _att_pool(x, w):
    s = jax.nn.softmax(jnp.einsum('btd,d->bt', x, w), axis=1)
    return jnp.sum(s[:, :, None] * x, axis=1)


def _decoder_kernel(h_ref, w1_ref, b1_ref, w2_ref, b2_ref, out_ref):
    h = h_ref[...]
    h = h @ w1_ref[...] + b1_ref[...]
    out_ref[...] = h @ w2_ref[...] + b2_ref[...]


def kernel(x_global, edge_index_global, x_sub, edge_index_sub, x_region, edge_index_region, params):
    h1g = jnp.zeros((NG, 1, 3), jnp.float32)
    h0g, _ = _run_block(x_global, edge_index_global, h1g, params['global'], NG)
    global_enc = h0g.reshape(BATCH, -1, OUT)
    global_enc = _att_pool(global_enc, params['pool_w']).reshape(BATCH, 1, OUT)
    h1s = jnp.zeros((NS, 1, 3), jnp.float32)
    _, h1so = _run_block(x_sub, edge_index_sub, h1s, params['sub'], NS)
    local_enc = h1so.reshape(-1, 16, C, 3).mean(axis=1)
    h0r, _ = _run_block(x_region, edge_index_region, local_enc, params['region'], NR)
    region_enc = h0r.reshape(BATCH, -1, OUT)
    h = jnp.concatenate([global_enc, region_enc], axis=1)
    h = _att_pool(h, params['whole_pool_w'])
    probs = pl.pallas_call(
        _decoder_kernel,
        out_shape=jax.ShapeDtypeStruct((BATCH, NCLASS), jnp.float32),
    )(h, params['dec_W1'], params['dec_b1'], params['dec_W2'], params['dec_b2'])
    return probs


# speedup vs baseline: 2.0507x; 2.0507x over previous
import functools

import jax
import jax.numpy as jnp
from jax import lax
from jax.experimental import pallas as pl
from jax.experimental.pallas import tpu as pltpu
from jax.experimental.pallas import tpu_sc as plsc

BATCH = 16
C = 16
OUT = 64
NCLASS = 15
RBASIS = 16
NG = 2048
NS = 16384
NR = 1024

NCORE = 2
NSUB = 16
NWORK = NCORE * NSUB
K = 128
ZROWS = 64

_CENTERS = [i * (4.0 / 15.0) for i in range(16)]


_MESH_CACHE = []


def _sc_mesh():
    if not _MESH_CACHE:
        m = object.__new__(plsc.VectorSubcoreMesh)
        object.__setattr__(m, "core_axis_name", "c")
        object.__setattr__(m, "subcore_axis_name", "s")
        object.__setattr__(m, "num_cores", NCORE)
        object.__setattr__(m, "num_subcores", NSUB)
        _MESH_CACHE.append(m)
    return _MESH_CACHE[0]


def _make_geom(n, e):
    ew = e // NWORK

    @functools.partial(
        pl.kernel,
        out_type=jax.ShapeDtypeStruct((e, 16), jnp.float32),
        mesh=_sc_mesh(),
        compiler_params=pltpu.CompilerParams(
            use_tc_tiling_on_sc=False, needs_layout_passes=False),
        scratch_types=[
            pltpu.VMEM((K,), jnp.int32),
            pltpu.VMEM((K,), jnp.int32),
            pltpu.VMEM((K, 16), jnp.float32),
            pltpu.VMEM((K, 16), jnp.float32),
            pltpu.VMEM((K, 16), jnp.float32),
            pltpu.SemaphoreType.DMA,
            pltpu.SemaphoreType.DMA,
        ],
    )
    def kern(x_h, src_h, dst_h, rel_h, sv, dv, xs, xd, ov, sm1, sm2):
        cid = lax.axis_index("c")
        sid = lax.axis_index("s")
        wid = cid * NSUB + sid

        def chunk(t, carry):
            e0 = wid * ew + t * K
            pltpu.sync_copy(src_h.at[pl.ds(e0, K)], sv)
            pltpu.sync_copy(dst_h.at[pl.ds(e0, K)], dv)
            cp1 = pltpu.async_copy(x_h.at[sv], xs, sm1)
            cp2 = pltpu.async_copy(x_h.at[dv], xd, sm2)
            cp1.wait()
            cp2.wait()

            def grp(g, c2):
                ov[g, :] = xd[g, :] - xs[g, :]
                return c2

            lax.fori_loop(0, K, grp, 0)
            pltpu.sync_copy(ov, rel_h.at[pl.ds(e0, K)])
            return carry

        lax.fori_loop(0, ew // K, chunk, 0)

    return kern


def _make_edge_noatt(n, e, ncols, has_p):
    ew = e // NWORK
    rpt = n // NSUB

    scratch = [
        pltpu.VMEM((K,), jnp.int32),
        pltpu.VMEM((K, ncols), jnp.float32),
        pltpu.VMEM((ZROWS, ncols), jnp.float32),
        pltpu.VMEM_SHARED((n, ncols), jnp.float32),
        pltpu.SemaphoreType.DMA,
    ]
    if has_p:
        scratch += [pltpu.VMEM((K,), jnp.int32), pltpu.VMEM((K, ncols), jnp.float32)]

    def body(refs):
        if has_p:
            g_h, p_h, src_h, dst_h, out_h, dv, gv, zb, acc, sm1, sv, pv = refs
        else:
            g_h, src_h, dst_h, out_h, dv, gv, zb, acc, sm1 = refs
            pv = sv = p_h = None
        cid = lax.axis_index("c")
        sid = lax.axis_index("s")
        wid = cid * NSUB + sid
        z16 = jnp.zeros((16,), jnp.float32)

        def zrow(r, carry):
            for cb in range(ncols // 16):
                zb[r, pl.ds(cb * 16, 16)] = z16
            return carry

        lax.fori_loop(0, ZROWS, zrow, 0)
        row0 = sid * rpt

        def zacc(i, carry):
            pltpu.sync_copy(zb, acc.at[pl.ds(row0 + i * ZROWS, ZROWS)])
            return carry

        lax.fori_loop(0, rpt // ZROWS, zacc, 0)
        plsc.subcore_barrier()

        def chunk(t, carry):
            e0 = wid * ew + t * K
            pltpu.sync_copy(dst_h.at[pl.ds(e0, K)], dv)
            pltpu.sync_copy(g_h.at[pl.ds(e0, K)], gv)
            pltpu.sync_copy(gv, acc.at[dv], add=True)
            if has_p:
                pltpu.sync_copy(src_h.at[pl.ds(e0, K)], sv)
                pltpu.async_copy(p_h.at[sv], pv, sm1).wait()
                pltpu.sync_copy(pv, acc.at[dv], add=True)
            return carry

        lax.fori_loop(0, ew // K, chunk, 0)
        plsc.subcore_barrier()
        pltpu.sync_copy(acc.at[pl.ds(row0, rpt)], out_h.at[pl.ds(cid * n + row0, rpt)])

    if has_p:
        def kern(g_h, p_h, src_h, dst_h, out_h, dv, gv, zb, acc, sm1, sv, pv):
            body((g_h, p_h, src_h, dst_h, out_h, dv, gv, zb, acc, sm1, sv, pv))
    else:
        def kern(g_h, src_h, dst_h, out_h, dv, gv, zb, acc, sm1):
            body((g_h, src_h, dst_h, out_h, dv, gv, zb, acc, sm1))

    return pl.kernel(
        kern,
        out_type=jax.ShapeDtypeStruct((2 * n, ncols), jnp.float32),
        mesh=_sc_mesh(),
        compiler_params=pltpu.CompilerParams(
            use_tc_tiling_on_sc=False, needs_layout_passes=False),
        scratch_types=scratch,
    )


def _make_edge_att(n, e, ncols):
    ew = e // NWORK
    rpt = n // NSUB

    @functools.partial(
        pl.kernel,
        out_type=jax.ShapeDtypeStruct((2 * n, ncols), jnp.float32),
        mesh=_sc_mesh(),
        compiler_params=pltpu.CompilerParams(
            use_tc_tiling_on_sc=False, needs_layout_passes=False),
        scratch_types=[
            pltpu.VMEM((K,), jnp.int32),
            pltpu.VMEM((K,), jnp.int32),
            pltpu.VMEM((K, ncols), jnp.float32),
            pltpu.VMEM((K, ncols), jnp.float32),
            pltpu.VMEM((K, 16), jnp.float32),
            pltpu.VMEM((K, ncols), jnp.float32),
            pltpu.VMEM((ZROWS, ncols), jnp.float32),
            pltpu.VMEM_SHARED((n, ncols), jnp.float32),
            pltpu.SemaphoreType.DMA,
            pltpu.SemaphoreType.DMA,
        ],
    )
    def kern(g_h, p_h, q_h, src_h, dst_h, out_h, dv, sv, gv, pv, qv, ov, zb, acc, sm1, sm2):
        cid = lax.axis_index("c")
        sid = lax.axis_index("s")
        wid = cid * NSUB + sid
        z16 = jnp.zeros((16,), jnp.float32)

        def zrow(r, carry):
            for cb in range(ncols // 16):
                zb[r, pl.ds(cb * 16, 16)] = z16
            return carry

        lax.fori_loop(0, ZROWS, zrow, 0)
        row0 = sid * rpt

        def zacc(i, carry):
            pltpu.sync_copy(zb, acc.at[pl.ds(row0 + i * ZROWS, ZROWS)])
            return carry

        lax.fori_loop(0, rpt // ZROWS, zacc, 0)
        plsc.subcore_barrier()
        iot = lax.iota(jnp.int32, 16)

        def chunk(t, carry):
            e0 = wid * ew + t * K
            pltpu.sync_copy(dst_h.at[pl.ds(e0, K)], dv)
            pltpu.sync_copy(src_h.at[pl.ds(e0, K)], sv)
            cp1 = pltpu.async_copy(p_h.at[sv], pv, sm1)
            cp2 = pltpu.async_copy(q_h.at[dv], qv, sm2)
            pltpu.sync_copy(g_h.at[pl.ds(e0, K)], gv)
            cp1.wait()
            cp2.wait()

            def grp(g, c2):
                rows = g * 16 + iot
                logit = jnp.zeros((16,), jnp.float32)
                for c in range(16):
                    cc = jnp.full((16,), c, jnp.int32)
                    tc = plsc.load_gather(gv, [rows, cc]) + plsc.load_gather(pv, [rows, cc])
                    logit = logit + plsc.load_gather(qv, [rows, cc]) * tc
                e16 = jnp.exp(logit * 0.25)
                for c in range(16):
                    cc = jnp.full((16,), c, jnp.int32)
                    plsc.store_scatter(ov, [rows, cc], e16)
                for c in range(16, ncols):
                    cc = jnp.full((16,), c, jnp.int32)
                    tc = plsc.load_gather(gv, [rows, cc]) + plsc.load_gather(pv, [rows, cc])
                    plsc.store_scatter(ov, [rows, cc], tc * e16)
                return c2

            lax.fori_loop(0, K // 16, grp, 0)
            pltpu.sync_copy(ov, acc.at[dv], add=True)
            return carry

        lax.fori_loop(0, ew // K, chunk, 0)
        plsc.subcore_barrier()
        pltpu.sync_copy(acc.at[pl.ds(row0, rpt)], out_h.at[pl.ds(cid * n + row0, rpt)])

    return kern


_EDGES = {NG: 32768, NS: 131072, NR: 16384}
with jax.set_mesh(None):
    _GEOM = {n: _make_geom(n, e) for n, e in _EDGES.items()}
    _NOATT = {
        NG: _make_edge_noatt(NG, _EDGES[NG], 80, False),
        NS: _make_edge_noatt(NS, _EDGES[NS], 80, False),
        NR: _make_edge_noatt(NR, _EDGES[NR], 80, True),
    }
    _ATT = {
        (NG, 80): _make_edge_att(NG, _EDGES[NG], 80),
        (NS, 80): _make_edge_att(NS, _EDGES[NS], 80),
        (NS, 64): _make_edge_att(NS, _EDGES[NS], 64),
        (NR, 80): _make_edge_att(NR, _EDGES[NR], 80),
    }


_ECH = 1024
_NCH = 512


def _full(shape):
    return pl.BlockSpec(shape, lambda i: tuple(0 for _ in shape))


def _chunked(shape):
    return pl.BlockSpec(shape, lambda i: (i,) + tuple(0 for _ in shape[1:]))


def _tc_edgeprep(rel, wstack, e, g3_cols, sub_mode):
    def kern(rel_ref, w_ref, g1_ref, g2_ref, g3_ref):
        centers = lax.broadcasted_iota(jnp.int32, (1, 16), 1).astype(jnp.float32) * (4.0 / 15.0)
        relc = rel_ref[...]
        rx = relc[:, 0:1]
        ry = relc[:, 1:2]
        rz = relc[:, 2:3]
        d2 = rx * rx + ry * ry + rz * rz
        r = jnp.sqrt(d2 + 1e-8)
        rbf = jnp.exp(-4.0 * (r - centers) ** 2)
        b = jnp.dot(rbf, w_ref[...], preferred_element_type=jnp.float32)
        ones = jnp.ones((rel_ref.shape[0], 16), jnp.float32)
        v11 = b[:, 16:32]
        g1_ref[...] = jnp.concatenate(
            [ones, b[:, 0:16], v11 * rx, v11 * ry, v11 * rz], axis=1)
        v12 = b[:, 64:80]
        g2_ref[...] = jnp.concatenate(
            [b[:, 32:48], b[:, 48:64], v12 * rx, v12 * ry, v12 * rz], axis=1)
        if sub_mode:
            v13 = b[:, 96:112]
            g3_ref[...] = jnp.concatenate(
                [b[:, 80:96], v13 * rx, v13 * ry, v13 * rz], axis=1)
        else:
            g3_ref[...] = b[:, 80:160]

    ws = wstack.shape[1]
    return pl.pallas_call(
        kern,
        grid=(e // _ECH,),
        in_specs=[_chunked((_ECH, 16)), _full((16, ws))],
        out_specs=[_chunked((_ECH, 80)), _chunked((_ECH, 80)), _chunked((_ECH, g3_cols))],
        out_shape=[
            jax.ShapeDtypeStruct((e, 80), jnp.float32),
            jax.ShapeDtypeStruct((e, 80), jnp.float32),
            jax.ShapeDtypeStruct((e, g3_cols), jnp.float32),
        ],
    )(rel, wstack)


def _h1_norm(m1, b1):
    nrm = jnp.sqrt(m1[:, 0:16] ** 2 + m1[:, 16:32] ** 2 + m1[:, 32:48] ** 2 + 1e-8)
    sc = jnp.maximum(nrm + b1, 0.0) / nrm
    return m1 * jnp.concatenate([sc, sc, sc], axis=1)


def _tc_np1(acc_a, acc_b, n, wnext, bdmix2, b0, b1, h1_0=None, bdws1=None):
    has_h1in = h1_0 is not None

    def kern(*refs):
        if has_h1in:
            (aa, ab, h1in_ref, wn_ref, bm_ref, bw_ref, b0_ref, b1_ref,
             h0_ref, h1_ref, p2_ref, q2_ref) = refs
        else:
            (aa, ab, wn_ref, bm_ref, b0_ref, b1_ref,
             h0_ref, h1_ref, p2_ref, q2_ref) = refs
        acc = aa[...] + ab[...]
        inv = 1.0 / (acc[:, 0:1] + 1e-9)
        h0 = jnp.maximum(acc[:, 16:32] * inv + b0_ref[...], 0.0)
        m1 = acc[:, 32:80] * inv
        if has_h1in:
            m1 = m1 + jnp.dot(h1in_ref[...], bw_ref[...], preferred_element_type=jnp.float32)
        h1 = _h1_norm(m1, b1_ref[...])
        x = jnp.dot(h0, wn_ref[...], preferred_element_type=jnp.float32)
        v1m = jnp.dot(h1, bm_ref[...], preferred_element_type=jnp.float32)
        h0_ref[...] = h0
        h1_ref[...] = h1
        p2_ref[...] = jnp.concatenate([x[:, 0:32], v1m], axis=1)
        q2_ref[...] = x[:, 32:48]

    ins = [_chunked((_NCH, 80)), _chunked((_NCH, 80))]
    args = [acc_a, acc_b]
    if has_h1in:
        ins.append(_chunked((_NCH, 48)))
        args.append(h1_0)
    ins += [_full((16, 48)), _full((48, 48))]
    args += [wnext, bdmix2]
    if has_h1in:
        ins.append(_full((48, 48)))
        args.append(bdws1)
    ins += [_full((1, 16)), _full((1, 16))]
    args += [b0, b1]
    return pl.pallas_call(
        kern,
        grid=(n // _NCH,),
        in_specs=ins,
        out_specs=[_chunked((_NCH, 16)), _chunked((_NCH, 48)),
                   _chunked((_NCH, 80)), _chunked((_NCH, 16))],
        out_shape=[
            jax.ShapeDtypeStruct((n, 16), jnp.float32),
            jax.ShapeDtypeStruct((n, 48), jnp.float32),
            jax.ShapeDtypeStruct((n, 80), jnp.float32),
            jax.ShapeDtypeStruct((n, 16), jnp.float32),
        ],
    )(*args)


def _tc_np2(acc_a, acc_b, h0_1, h1_1, n, ws0, bdws1, b0, b1, wnext, bdmix3, p3_cols):
    sub_mode = bdmix3 is not None

    def kern(*refs):
        if sub_mode:
            (aa, ab, h0_ref_in, h1_ref_in, ws0_ref, bw_ref, b0_ref, b1_ref,
             wn_ref, bm_ref, h0o, h1o, p3_ref, q3_ref) = refs
        else:
            (aa, ab, h0_ref_in, h1_ref_in, ws0_ref, bw_ref, b0_ref, b1_ref,
             wn_ref, h0o, h1o, p3_ref, q3_ref) = refs
        acc = aa[...] + ab[...]
        inv = 1.0 / (acc[:, 0:1] + 1e-9)
        h0p = h0_ref_in[...]
        h1p = h1_ref_in[...]
        h0 = jnp.maximum(
            acc[:, 16:32] * inv
            + jnp.dot(h0p, ws0_ref[...], preferred_element_type=jnp.float32)
            + b0_ref[...], 0.0)
        m1 = acc[:, 32:80] * inv + jnp.dot(h1p, bw_ref[...], preferred_element_type=jnp.float32)
        h1 = _h1_norm(m1, b1_ref[...])
        x = jnp.dot(h0, wn_ref[...], preferred_element_type=jnp.float32)
        h0o[...] = h0
        h1o[...] = h1
        if sub_mode:
            v1m = jnp.dot(h1, bm_ref[...], preferred_element_type=jnp.float32)
            p3_ref[...] = jnp.concatenate([x[:, 0:16], v1m], axis=1)
            q3_ref[...] = x[:, 16:32]
        else:
            p3_ref[...] = x[:, 0:80]
            q3_ref[...] = x[:, 80:96]

    wn_cols = 32 if sub_mode else 96
    ins = [_chunked((_NCH, 80)), _chunked((_NCH, 80)), _chunked((_NCH, 16)),
           _chunked((_NCH, 48)), _full((16, 16)), _full((48, 48)),
           _full((1, 16)), _full((1, 16)), _full((16, wn_cols))]
    args = [acc_a, acc_b, h0_1, h1_1, ws0, bdws1, b0, b1, wnext]
    if sub_mode:
        ins.append(_full((48, 48)))
        args.append(bdmix3)
    return pl.pallas_call(
        kern,
        grid=(n // _NCH,),
        in_specs=ins,
        out_specs=[_chunked((_NCH, 16)), _chunked((_NCH, 48)),
                   _chunked((_NCH, p3_cols)), _chunked((_NCH, 16))],
        out_shape=[
            jax.ShapeDtypeStruct((n, 16), jnp.float32),
            jax.ShapeDtypeStruct((n, 48), jnp.float32),
            jax.ShapeDtypeStruct((n, p3_cols), jnp.float32),
            jax.ShapeDtypeStruct((n, 16), jnp.float32),
        ],
    )(*args)


def _tc_np3_h0(acc_a, acc_b, h0_2, n, ws0, b0):
    def kern(aa, ab, h0in, ws0_ref, b0_ref, out_ref):
        acc = aa[...] + ab[...]
        inv = 1.0 / (acc[:, 0:1] + 1e-9)
        out_ref[...] = jnp.maximum(
            acc[:, 16:80] * inv
            + jnp.dot(h0in[...], ws0_ref[...], preferred_element_type=jnp.float32)
            + b0_ref[...], 0.0)

    return pl.pallas_call(
        kern,
        grid=(n // _NCH,),
        in_specs=[_chunked((_NCH, 80)), _chunked((_NCH, 80)), _chunked((_NCH, 16)),
                  _full((16, 64)), _full((1, 64))],
        out_specs=_chunked((_NCH, 64)),
        out_shape=jax.ShapeDtypeStruct((n, 64), jnp.float32),
    )(acc_a, acc_b, h0_2, ws0, b0)


def _tc_np3_sub(acc_a, acc_b, h1_2, n, bdws1, b1, amean, bdmixr1):
    def kern(aa, ab, h1in, bw_ref, b1_ref, am_ref, bm_ref, loc_ref, p1r_ref):
        acc = aa[...] + ab[...]
        inv = 1.0 / (acc[:, 0:1] + 1e-9)
        m1 = acc[:, 16:64] * inv + jnp.dot(h1in[...], bw_ref[...], preferred_element_type=jnp.float32)
        h1 = _h1_norm(m1, b1_ref[...])
        loc = jnp.dot(am_ref[...], h1, preferred_element_type=jnp.float32)
        loc_ref[...] = loc
        v1m = jnp.dot(loc, bm_ref[...], preferred_element_type=jnp.float32)
        p1r_ref[...] = jnp.concatenate([jnp.zeros((loc.shape[0], 32), jnp.float32), v1m], axis=1)

    gsz = _NCH // 16
    return pl.pallas_call(
        kern,
        grid=(n // _NCH,),
        in_specs=[_chunked((_NCH, 64)), _chunked((_NCH, 64)), _chunked((_NCH, 48)),
                  _full((48, 48)), _full((1, 16)), _full((gsz, _NCH)), _full((48, 48))],
        out_specs=[_chunked((gsz, 48)), _chunked((gsz, 80))],
        out_shape=[
            jax.ShapeDtypeStruct((n // 16, 48), jnp.float32),
            jax.ShapeDtypeStruct((n // 16, 80), jnp.float32),
        ],
    )(acc_a, acc_b, h1_2, bdws1, b1, amean, bdmixr1)


def _tc_pool_decode(h0g, h0r, pw, wpw, w1, b1, w2, b2):
    def kern(hg_ref, hr_ref, pw_ref, wpw_ref, w1_ref, b1_ref, w2_ref, b2_ref, out_ref):
        hg = hg_ref[...]
        lg = jnp.dot(hg, pw_ref[...], preferred_element_type=jnp.float32)
        lg = lg - jnp.max(lg, axis=0, keepdims=True)
        eg = jnp.exp(lg)
        ge = jnp.sum(eg * hg, axis=0, keepdims=True) / jnp.sum(eg, axis=0, keepdims=True)
        hr = hr_ref[...]
        l2r = jnp.dot(hr, wpw_ref[...], preferred_element_type=jnp.float32)
        l2g = jnp.dot(ge, wpw_ref[...], preferred_element_type=jnp.float32)
        m = jnp.maximum(jnp.max(l2r, axis=0, keepdims=True), l2g)
        er = jnp.exp(l2r - m)
        eg2 = jnp.exp(l2g - m)
        s = jnp.sum(er, axis=0, keepdims=True) + eg2
        hp = (jnp.sum(er * hr, axis=0, keepdims=True) + eg2 * ge) / s
        o = jnp.dot(hp, w1_ref[...], preferred_element_type=jnp.float32) + b1_ref[...]
        o = jnp.dot(o, w2_ref[...], preferred_element_type=jnp.float32) + b2_ref[...]
        out_ref[pl.ds(pl.program_id(0), 1), :] = o

    return pl.pallas_call(
        kern,
        grid=(BATCH,),
        in_specs=[_chunked((128, 64)), _chunked((64, 64)), _full((64, 1)), _full((64, 1)),
                  _full((64, 64)), _full((1, 64)), _full((64, NCLASS)), _full((1, NCLASS))],
        out_specs=_full((BATCH, NCLASS)),
        out_shape=jax.ShapeDtypeStruct((BATCH, NCLASS), jnp.float32),
    )(h0g, h0r, pw, wpw, w1, b1, w2, b2)


def _bd3(w):
    z = jnp.zeros_like(w)
    return jnp.concatenate([
        jnp.concatenate([w, z, z], axis=1),
        jnp.concatenate([z, w, z], axis=1),
        jnp.concatenate([z, z, w], axis=1),
    ], axis=0)


def _block(x, ei, plist, n, e, sub_mode, h1_0=None, p1=None):
    src = ei[0].astype(jnp.int32)
    dst = ei[1].astype(jnp.int32)
    x16 = jnp.pad(x, ((0, 0), (0, 13)))
    rel = _GEOM[n](x16, src, dst)

    p1w, p2w, p3w = plist
    wst = [p1w['Wv0_r'], p1w['Wv1_r'], p2w['Wk_r'], p2w['Wv0_r'], p2w['Wv1_r'], p3w['Wk_r']]
    wst.append(p3w['Wv1_r'] if sub_mode else p3w['Wv0_r'])
    wstack = jnp.concatenate(wst, axis=1)
    g3_cols = 64 if sub_mode else 80
    g1, g2, g3 = _tc_edgeprep(rel, wstack, e, g3_cols, sub_mode)

    if p1 is None:
        acc = _NOATT[n](g1, src, dst)
    else:
        acc = _NOATT[n](g1, p1, src, dst)
    wnext = jnp.concatenate([p2w['Wk_h'], p2w['Wv0_h'], p2w['Wq']], axis=1)
    h0_1, h1_1, p2t, q2t = _tc_np1(
        acc[:n], acc[n:], n, wnext, _bd3(p2w['Wv1_mix']),
        p1w['b0'][None, :], p1w['b1'][None, :],
        h1_0=h1_0, bdws1=None if h1_0 is None else _bd3(p1w['Ws1']))

    acc = _ATT[(n, 80)](g2, p2t, q2t, src, dst)
    if sub_mode:
        wnext3 = jnp.concatenate([p3w['Wk_h'], p3w['Wq']], axis=1)
        bdmix3 = _bd3(p3w['Wv1_mix'])
    else:
        wnext3 = jnp.concatenate([p3w['Wk_h'], p3w['Wv0_h'], p3w['Wq']], axis=1)
        bdmix3 = None
    h0_2, h1_2, p3t, q3t = _tc_np2(
        acc[:n], acc[n:], h0_1, h1_1, n, p2w['Ws0'], _bd3(p2w['Ws1']),
        p2w['b0'][None, :], p2w['b1'][None, :], wnext3, bdmix3, g3_cols)

    acc = _ATT[(n, g3_cols)](g3, p3t, q3t, src, dst)
    if sub_mode:
        return acc, h1_2
    return _tc_np3_h0(acc[:n], acc[n:], h0_2, n, p3w['Ws0'], p3w['b0'][None, :])


def kernel(x_global, edge_index_global, x_sub, edge_index_sub,
           x_region, edge_index_region, params):
    pg, ps, pr = params['global'], params['sub'], params['region']

    h0g = _block(x_global, edge_index_global, pg, NG, 32768, False)

    acc_s3, h1s_2 = _block(x_sub, edge_index_sub, ps, NS, 131072, True)
    amean = jnp.repeat(jnp.eye(_NCH // 16, dtype=jnp.float32), 16, axis=1) / 16.0
    local, p1r = _tc_np3_sub(
        acc_s3[:NS], acc_s3[NS:], h1s_2, NS, _bd3(ps[2]['Ws1']),
        ps[2]['b1'][None, :], amean, _bd3(pr[0]['Wv1_mix']))

    h0r = _block(x_region, edge_index_region, pr, NR, 16384, False,
                 h1_0=local, p1=p1r)

    return _tc_pool_decode(
        h0g, h0r, params['pool_w'][:, None], params['whole_pool_w'][:, None],
        params['dec_W1'], params['dec_b1'][None, :],
        params['dec_W2'], params['dec_b2'][None, :])
